# baseline (speedup 1.0000x reference)
import jax
import jax.numpy as jnp
import numpy as np
from jax.experimental import pallas as pl
from jax.experimental.pallas import tpu as pltpu

_ALPHA = np.float32(1.0 / 100.0)
_DECAY = np.float32(1.0 - (1.0 / 100.0))
_THR = np.float32(0.58)
_WTA = np.float32(0.85)
_K = 2

_SUB = 8
_BT = 256


def _body(spk_ref, mask_ref, cov_ref, nmda_ref, iota_ref, iota2_ref):
    lanes = nmda_ref.shape[1]
    n = _SUB * lanes

    @pl.when(pl.program_id(0) == 0)
    def _init():
        nmda_ref[...] = jnp.zeros_like(nmda_ref)
        r = jax.lax.broadcasted_iota(jnp.int32, (_SUB, lanes), 0)
        c = jax.lax.broadcasted_iota(jnp.int32, (_SUB, lanes), 1)
        iota_ref[...] = r * lanes + c
        iota2_ref[...] = jax.lax.broadcasted_iota(jnp.int32, (_SUB, n), 1)

    iota = iota_ref[...]
    big = jnp.int32(1 << 30)
    covc = np.float32(_K / (_SUB * 1.0)) / np.float32(1.0 * lanes)

    nmda = nmda_ref[...]
    scores = []
    mxs = []
    for j in range(_BT):
        nmda = _DECAY * nmda + _ALPHA * spk_ref[j]
        scores.append(nmda * _WTA)
        mxs.append(jnp.max(nmda))
    nmda_ref[...] = nmda

    m1s = [_WTA * mx for mx in mxs]
    i1s = [jnp.min(jnp.where(s == m1, iota, big))
           for s, m1 in zip(scores, m1s)]

    sc2s = [jnp.where(iota == i1, np.float32(-1.0), s)
            for s, i1 in zip(scores, i1s)]
    m2s = [jnp.max(s2) for s2 in sc2s]

    i2s = [jnp.min(jnp.where(s2 == m2, iota, big))
           for s2, m2 in zip(sc2s, m2s)]

    iota_n = iota2_ref[...]
    fires = [jnp.where(mx >= _THR, np.float32(1.0), np.float32(0.0))
             for mx in mxs]
    for g in range(_BT // _SUB):
        i1b = jnp.stack(i1s[g * _SUB:(g + 1) * _SUB]).reshape(_SUB, 1)
        i2b = jnp.stack(i2s[g * _SUB:(g + 1) * _SUB]).reshape(_SUB, 1)
        fb = jnp.stack(fires[g * _SUB:(g + 1) * _SUB]).reshape(_SUB, 1)
        sel = (iota_n == i1b) | (iota_n == i2b)
        mask_ref[g] = jnp.where(sel, fb, np.float32(0.0))
        cov_ref[pl.ds(g * _SUB, _SUB)] = jnp.broadcast_to(
            fb * covc, (_SUB, 128))


def kernel(spikes):
    t, n = spikes.shape
    lanes = n // _SUB
    ng = _BT // _SUB
    spk3 = spikes.reshape(t, _SUB, lanes)
    mask3, cov2 = pl.pallas_call(
        _body,
        grid=(t // _BT,),
        in_specs=[pl.BlockSpec((_BT, _SUB, lanes), lambda i: (i, 0, 0))],
        out_specs=[
            pl.BlockSpec((ng, _SUB, n), lambda i: (i, 0, 0)),
            pl.BlockSpec((_BT, 128), lambda i: (i, 0)),
        ],
        out_shape=[
            jax.ShapeDtypeStruct((t // _SUB, _SUB, n), jnp.float32),
            jax.ShapeDtypeStruct((t, 128), jnp.float32),
        ],
        scratch_shapes=[
            pltpu.VMEM((_SUB, lanes), jnp.float32),
            pltpu.VMEM((_SUB, lanes), jnp.int32),
            pltpu.VMEM((_SUB, n), jnp.int32),
        ],
    )(spk3)
    return mask3.reshape(t, n), cov2[:, 0]


# speedup vs baseline: 1.0601x; 1.0601x over previous
import jax
import jax.numpy as jnp
import numpy as np
from jax.experimental import pallas as pl
from jax.experimental.pallas import tpu as pltpu

_ALPHA = np.float32(1.0 / 100.0)
_DECAY = np.float32(1.0 - (1.0 / 100.0))
_THR = np.float32(0.58)
_WTA = np.float32(0.85)
_K = 2

_SUB = 8
_BT = 128


def _body(spk_ref, mask_ref, cov_ref, nmda_ref, iota_ref, iota2_ref):
    lanes = nmda_ref.shape[1]
    n = _SUB * lanes

    @pl.when(pl.program_id(0) == 0)
    def _init():
        nmda_ref[...] = jnp.zeros_like(nmda_ref)
        r = jax.lax.broadcasted_iota(jnp.int32, (_SUB, lanes), 0)
        c = jax.lax.broadcasted_iota(jnp.int32, (_SUB, lanes), 1)
        iota_ref[...] = (r * lanes + c).astype(jnp.float32)
        iota2_ref[...] = jax.lax.broadcasted_iota(
            jnp.int32, (_SUB, n), 1).astype(jnp.float32)

    iota = iota_ref[...]
    big = np.float32(np.inf)
    covc = np.float32(_K / (_SUB * 1.0)) / np.float32(1.0 * lanes)

    nmda = nmda_ref[...]
    scores = []
    mxs = []
    for j in range(_BT):
        nmda = _DECAY * nmda + _ALPHA * spk_ref[j]
        scores.append(nmda * _WTA)
        mxs.append(jnp.max(nmda))
    nmda_ref[...] = nmda

    m1s = [_WTA * mx for mx in mxs]
    i1s = [jnp.min(jnp.where(s == m1, iota, big))
           for s, m1 in zip(scores, m1s)]

    sc2s = [jnp.where(iota == i1, np.float32(-1.0), s)
            for s, i1 in zip(scores, i1s)]
    m2s = [jnp.max(s2) for s2 in sc2s]

    i2s = [jnp.min(jnp.where(s2 == m2, iota, big))
           for s2, m2 in zip(sc2s, m2s)]

    iota_n = iota2_ref[...]
    fires = [jnp.where(mx >= _THR, np.float32(1.0), np.float32(0.0))
             for mx in mxs]
    for g in range(_BT // _SUB):
        i1b = jnp.stack(i1s[g * _SUB:(g + 1) * _SUB]).reshape(_SUB, 1)
        i2b = jnp.stack(i2s[g * _SUB:(g + 1) * _SUB]).reshape(_SUB, 1)
        fb = jnp.stack(fires[g * _SUB:(g + 1) * _SUB]).reshape(_SUB, 1)
        sel = (iota_n == i1b) | (iota_n == i2b)
        mask_ref[g] = jnp.where(sel, fb, np.float32(0.0))
        cov_ref[pl.ds(g * _SUB, _SUB)] = jnp.broadcast_to(
            fb * covc, (_SUB, 128))


def kernel(spikes):
    t, n = spikes.shape
    lanes = n // _SUB
    ng = _BT // _SUB
    spk3 = spikes.reshape(t, _SUB, lanes)
    mask3, cov2 = pl.pallas_call(
        _body,
        grid=(t // _BT,),
        in_specs=[pl.BlockSpec((_BT, _SUB, lanes), lambda i: (i, 0, 0))],
        out_specs=[
            pl.BlockSpec((ng, _SUB, n), lambda i: (i, 0, 0)),
            pl.BlockSpec((_BT, 128), lambda i: (i, 0)),
        ],
        out_shape=[
            jax.ShapeDtypeStruct((t // _SUB, _SUB, n), jnp.float32),
            jax.ShapeDtypeStruct((t, 128), jnp.float32),
        ],
        scratch_shapes=[
            pltpu.VMEM((_SUB, lanes), jnp.float32),
            pltpu.VMEM((_SUB, lanes), jnp.float32),
            pltpu.VMEM((_SUB, n), jnp.float32),
        ],
    )(spk3)
    return mask3.reshape(t, n), cov2[:, 0]
